# baseline (speedup 1.0000x reference)
import functools

import jax
import jax.numpy as jnp
from jax import lax
from jax.experimental import pallas as pl
from jax.experimental.pallas import tpu as pltpu
from jax.experimental.pallas import tpu_sc as plsc

B = 16384
N_CONT = 13
N_CAT = 26
VOCAB = 100000
D = 32
N_TGT = 2

NC, NS = 2, 16
NW = NC * NS
TOTAL_ROWS = B * N_CAT
ROWS_PER_W = TOTAL_ROWS // NW
CHUNK = 1024
CHUNKS_PER_W = ROWS_PER_W // CHUNK
IDX_MINOR = 128
IDX_ROWS_PER_CHUNK = CHUNK // IDX_MINOR


def _sc_gather(table_flat, gidx2d):
    mesh = plsc.VectorSubcoreMesh(core_axis_name="c", subcore_axis_name="s")

    @functools.partial(
        pl.kernel,
        mesh=mesh,
        compiler_params=pltpu.CompilerParams(use_tc_tiling_on_sc=False),
        out_type=jax.ShapeDtypeStruct((TOTAL_ROWS, D), jnp.float32),
        scratch_types=[
            pltpu.VMEM((CHUNK,), jnp.int32),
            pltpu.VMEM((CHUNK, D), jnp.float32),
            pltpu.SemaphoreType.DMA,
        ],
    )
    def k(table_hbm, gidx_hbm, out_hbm, idx_v, rows_v, sem):
        wid = lax.axis_index("s") * NC + lax.axis_index("c")

        def body(i, carry):
            row0 = wid * ROWS_PER_W + i * CHUNK
            pltpu.sync_copy(gidx_hbm.at[pl.ds(row0, CHUNK)], idx_v)
            pltpu.async_copy(table_hbm.at[idx_v], rows_v, sem).wait()
            pltpu.sync_copy(rows_v, out_hbm.at[pl.ds(row0, CHUNK)])
            return carry

        lax.fori_loop(0, CHUNKS_PER_W, body, 0, unroll=False)

    return k(table_flat, gidx2d)


BB = 512


def _tc_body(x_ref, we_ref, bf_ref, cat_ref, tgt_ref, ctx_ref, cls_ref):
    x = x_ref[...]
    cont = jnp.dot(x, we_ref[...],
                   preferred_element_type=jnp.float32,
                   precision=lax.Precision.HIGHEST) + bf_ref[...]
    ctx_ref[:, : N_CONT * D] = cont
    ctx_ref[:, N_CONT * D:] = cat_ref[...]
    cls_ref[...] = jnp.broadcast_to(tgt_ref[...], (BB, N_TGT * D))


def _tc_assemble(x_cont, WE, bflat, cat_rows2d, tgt_flat):
    grid = (B // BB,)
    ctx2d, cls2d = pl.pallas_call(
        _tc_body,
        grid=grid,
        in_specs=[
            pl.BlockSpec((BB, N_CONT), lambda i: (i, 0)),
            pl.BlockSpec((N_CONT, N_CONT * D), lambda i: (0, 0)),
            pl.BlockSpec((1, N_CONT * D), lambda i: (0, 0)),
            pl.BlockSpec((BB, N_CAT * D), lambda i: (i, 0)),
            pl.BlockSpec((1, N_TGT * D), lambda i: (0, 0)),
        ],
        out_specs=[
            pl.BlockSpec((BB, (N_CONT + N_CAT) * D), lambda i: (i, 0)),
            pl.BlockSpec((BB, N_TGT * D), lambda i: (i, 0)),
        ],
        out_shape=[
            jax.ShapeDtypeStruct((B, (N_CONT + N_CAT) * D), jnp.float32),
            jax.ShapeDtypeStruct((B, N_TGT * D), jnp.float32),
        ],
    )(x_cont, WE, bflat, cat_rows2d, tgt_flat)
    return ctx2d, cls2d


def kernel(x_cat, x_cont, lin_W, lin_b, cat_tables, tgt):
    table_flat = cat_tables.reshape(N_CAT * VOCAB, D)
    gidx = x_cat + (jnp.arange(N_CAT, dtype=jnp.int32) * VOCAB)[None, :]
    gidx1d = gidx.reshape(TOTAL_ROWS)
    eye = jnp.eye(N_CONT, dtype=jnp.float32)
    WE = (eye[:, :, None] * lin_W[:, None, :]).reshape(N_CONT, N_CONT * D)
    bflat = lin_b.reshape(1, N_CONT * D)
    tgt_flat = tgt.reshape(1, N_TGT * D)

    cat_rows = _sc_gather(table_flat, gidx1d)
    cat_rows2d = cat_rows.reshape(B, N_CAT * D)

    ctx2d, cls2d = _tc_assemble(x_cont, WE, bflat, cat_rows2d, tgt_flat)

    context = ctx2d.reshape(B, N_CONT + N_CAT, D)
    class_embeddings = cls2d.reshape(B, N_TGT, D)
    return (class_embeddings, context)


# speedup vs baseline: 2.9290x; 2.9290x over previous
import functools

import jax
import jax.numpy as jnp
from jax import lax
from jax.experimental import pallas as pl
from jax.experimental.pallas import tpu as pltpu
from jax.experimental.pallas import tpu_sc as plsc

B = 16384
N_CONT = 13
N_CAT = 26
VOCAB = 100000
D = 32
N_TGT = 2

NC, NS = 2, 16
NW = NC * NS
ROWS = N_CAT * D
ROWS_PER_W = ROWS // NW
IDX_HALF = B // 2


def _sc_gather_t(tabT, xcT_flat):
    mesh = plsc.VectorSubcoreMesh(core_axis_name="c", subcore_axis_name="s")

    @functools.partial(
        pl.kernel,
        mesh=mesh,
        compiler_params=pltpu.CompilerParams(
            use_tc_tiling_on_sc=True, needs_layout_passes=False),
        out_type=jax.ShapeDtypeStruct((ROWS, B), jnp.float32),
        scratch_types=[
            pltpu.VMEM((1, VOCAB), jnp.float32),
            pltpu.VMEM((IDX_HALF,), jnp.int32),
            pltpu.VMEM((1, B), jnp.float32),
        ],
    )
    def k(tab_hbm, xc_hbm, out_hbm, seg_v, idx_v, row_v):
        wid = lax.axis_index("s") * NC + lax.axis_index("c")
        zeros16 = jnp.zeros((16,), jnp.int32)

        def pair_body(p, c0):
            fd = wid * ROWS_PER_W + p
            f = fd // D
            pltpu.sync_copy(tab_hbm.at[pl.ds(fd, 1), :], seg_v)

            def half_body(h, c1):
                pltpu.sync_copy(
                    xc_hbm.at[pl.ds(f * B + h * IDX_HALF, IDX_HALF)], idx_v)

                def vec_body(j, c2):
                    idx16 = idx_v[pl.ds(j * 16, 16)]
                    vals = plsc.load_gather(seg_v, [zeros16, idx16])
                    row_v[0, pl.ds(h * IDX_HALF + j * 16, 16)] = vals
                    return c2

                lax.fori_loop(0, IDX_HALF // 16, vec_body, c1, unroll=8)
                return c1

            lax.fori_loop(0, 2, half_body, c0, unroll=False)
            pltpu.sync_copy(row_v, out_hbm.at[pl.ds(fd, 1), :])
            return c0

        lax.fori_loop(0, ROWS_PER_W, pair_body, 0, unroll=False)

    return k(tabT, xcT_flat)


BC = 2048


def _tc_body(x_ref, wt_ref, bt_ref, cat_ref, tgtT_ref, ctx_ref, cls_ref):
    for i in range(N_CONT):
        ctx_ref[pl.ds(i * D, D), :] = (
            wt_ref[:, i:i + 1] * x_ref[i:i + 1, :] + bt_ref[:, i:i + 1])
    ctx_ref[pl.ds(N_CONT * D, ROWS), :] = cat_ref[...]
    for t in range(N_TGT):
        cls_ref[pl.ds(t * D, D), :] = jnp.broadcast_to(
            tgtT_ref[:, t:t + 1], (D, BC))


def _tc_assemble(x_contT, WT, bT, catT, tgtT):
    grid = (B // BC,)
    ctxT, clsT = pl.pallas_call(
        _tc_body,
        grid=grid,
        in_specs=[
            pl.BlockSpec((N_CONT, BC), lambda i: (0, i)),
            pl.BlockSpec((D, N_CONT), lambda i: (0, 0)),
            pl.BlockSpec((D, N_CONT), lambda i: (0, 0)),
            pl.BlockSpec((ROWS, BC), lambda i: (0, i)),
            pl.BlockSpec((D, N_TGT), lambda i: (0, 0)),
        ],
        out_specs=[
            pl.BlockSpec(((N_CONT + N_CAT) * D, BC), lambda i: (0, i)),
            pl.BlockSpec((N_TGT * D, BC), lambda i: (0, i)),
        ],
        out_shape=[
            jax.ShapeDtypeStruct(((N_CONT + N_CAT) * D, B), jnp.float32),
            jax.ShapeDtypeStruct((N_TGT * D, B), jnp.float32),
        ],
    )(x_contT, WT, bT, catT, tgtT)
    return ctxT, clsT


def kernel(x_cat, x_cont, lin_W, lin_b, cat_tables, tgt):
    tabT = jnp.transpose(cat_tables, (0, 2, 1)).reshape(ROWS, VOCAB)
    xcT_flat = jnp.transpose(x_cat, (1, 0)).reshape(N_CAT * B)
    x_contT = jnp.transpose(x_cont, (1, 0))
    WT = jnp.transpose(lin_W, (1, 0))
    bT = jnp.transpose(lin_b, (1, 0))
    tgtT = jnp.transpose(tgt, (1, 0))

    catT = _sc_gather_t(tabT, xcT_flat)
    ctxT, clsT = _tc_assemble(x_contT, WT, bT, catT, tgtT)

    context = jnp.transpose(
        ctxT.reshape(N_CONT + N_CAT, D, B), (2, 0, 1))
    class_embeddings = jnp.transpose(clsT.reshape(N_TGT, D, B), (2, 0, 1))
    return (class_embeddings, context)


# speedup vs baseline: 3.4764x; 1.1869x over previous
import functools

import jax
import jax.numpy as jnp
from jax import lax
from jax.experimental import pallas as pl
from jax.experimental.pallas import tpu as pltpu
from jax.experimental.pallas import tpu_sc as plsc

B = 16384
N_CONT = 13
N_CAT = 26
VOCAB = 100000
D = 32
N_TGT = 2

NC, NS = 2, 16
NW = NC * NS
ROWS = N_CAT * D
ROWS_PER_W = ROWS // NW
CONT_ROWS = N_CONT * D
CTX_ROWS = CONT_ROWS + ROWS
ROW_HALF = B // 2


def _sc_gather_t(tabT, xcT_flat):
    mesh = plsc.VectorSubcoreMesh(core_axis_name="c", subcore_axis_name="s")

    @functools.partial(
        pl.kernel,
        mesh=mesh,
        compiler_params=pltpu.CompilerParams(
            use_tc_tiling_on_sc=True, needs_layout_passes=False),
        out_type=jax.ShapeDtypeStruct((CTX_ROWS, B), jnp.float32),
        scratch_types=[
            pltpu.VMEM((1, VOCAB), jnp.float32),
            pltpu.VMEM((B,), jnp.int32),
            pltpu.VMEM((1, ROW_HALF), jnp.float32),
        ],
    )
    def k(tab_hbm, xc_hbm, out_hbm, seg_v, idx_v, row_v):
        wid = lax.axis_index("s") * NC + lax.axis_index("c")
        zeros16 = jnp.zeros((16,), jnp.int32)

        def pair_body(p, prev_f):
            fd = wid * ROWS_PER_W + p
            f = fd // D

            @pl.when(f != prev_f)
            def _load_idx():
                pltpu.sync_copy(xc_hbm.at[pl.ds(f * B, B)], idx_v)

            pltpu.sync_copy(tab_hbm.at[pl.ds(fd, 1), :], seg_v)

            def half_body(h, c1):
                def vec_body(j, c2):
                    idx16 = idx_v[pl.ds(h * ROW_HALF + j * 16, 16)]
                    vals = plsc.load_gather(seg_v, [zeros16, idx16])
                    row_v[0, pl.ds(j * 16, 16)] = vals
                    return c2

                lax.fori_loop(0, ROW_HALF // 16, vec_body, c1, unroll=8)
                pltpu.sync_copy(
                    row_v,
                    out_hbm.at[pl.ds(CONT_ROWS + fd, 1),
                               pl.ds(h * ROW_HALF, ROW_HALF)])
                return c1

            lax.fori_loop(0, 2, half_body, 0, unroll=False)
            return f

        lax.fori_loop(0, ROWS_PER_W, pair_body, -1, unroll=False)

    return k(tabT, xcT_flat)


BC = 2048


def _tc_body(x_ref, wt_ref, bt_ref, tgtT_ref, ctx_in_ref, ctx_ref, cls_ref):
    del ctx_in_ref
    for i in range(N_CONT):
        ctx_ref[pl.ds(i * D, D), :] = (
            wt_ref[:, i:i + 1] * x_ref[i:i + 1, :] + bt_ref[:, i:i + 1])
    for t in range(N_TGT):
        cls_ref[pl.ds(t * D, D), :] = jnp.broadcast_to(
            tgtT_ref[:, t:t + 1], (D, BC))


def _tc_assemble(x_contT, WT, bT, tgtT, ctx_partial):
    grid = (B // BC,)
    ctxT, clsT = pl.pallas_call(
        _tc_body,
        grid=grid,
        in_specs=[
            pl.BlockSpec((N_CONT, BC), lambda i: (0, i)),
            pl.BlockSpec((D, N_CONT), lambda i: (0, 0)),
            pl.BlockSpec((D, N_CONT), lambda i: (0, 0)),
            pl.BlockSpec((D, N_TGT), lambda i: (0, 0)),
            pl.BlockSpec((8, 128), lambda i: (0, 0)),
        ],
        out_specs=[
            pl.BlockSpec((CONT_ROWS, BC), lambda i: (0, i)),
            pl.BlockSpec((N_TGT * D, BC), lambda i: (0, i)),
        ],
        out_shape=[
            jax.ShapeDtypeStruct((CTX_ROWS, B), jnp.float32),
            jax.ShapeDtypeStruct((N_TGT * D, B), jnp.float32),
        ],
        input_output_aliases={4: 0},
    )(x_contT, WT, bT, tgtT, ctx_partial)
    return ctxT, clsT


def kernel(x_cat, x_cont, lin_W, lin_b, cat_tables, tgt):
    tabT = jnp.transpose(cat_tables, (0, 2, 1)).reshape(ROWS, VOCAB)
    xcT_flat = jnp.transpose(x_cat, (1, 0)).reshape(N_CAT * B)
    x_contT = jnp.transpose(x_cont, (1, 0))
    WT = jnp.transpose(lin_W, (1, 0))
    bT = jnp.transpose(lin_b, (1, 0))
    tgtT = jnp.transpose(tgt, (1, 0))

    ctx_partial = _sc_gather_t(tabT, xcT_flat)
    ctxT, clsT = _tc_assemble(x_contT, WT, bT, tgtT, ctx_partial)

    context = jnp.transpose(
        ctxT.reshape(N_CONT + N_CAT, D, B), (2, 0, 1))
    class_embeddings = jnp.transpose(clsT.reshape(N_TGT, D, B), (2, 0, 1))
    return (class_embeddings, context)
